# baseline (speedup 1.0000x reference)
import functools

import jax
import jax.numpy as jnp
from jax import lax
from jax.experimental import pallas as pl
from jax.experimental.pallas import tpu as pltpu
from jax.experimental.pallas import tpu_sc as plsc

IDXW = 128


@functools.lru_cache(maxsize=None)
def _build(n_tokens: int, vocab: int, hidden: int):
    info = plsc.get_sparse_core_info()
    nc, ns = info.num_cores, info.num_subcores
    nw = nc * ns

    n_rows = n_tokens // IDXW
    rows_per_w = n_rows // nw
    rpc = 4
    n_chunks = rows_per_w // rpc
    chunk_tokens = rpc * IDXW

    assert n_rows % nw == 0 and rows_per_w % rpc == 0

    mesh = plsc.VectorSubcoreMesh(core_axis_name="c", subcore_axis_name="s")

    @functools.partial(
        pl.kernel,
        mesh=mesh,
        compiler_params=pltpu.CompilerParams(use_tc_tiling_on_sc=False),
        out_type=jax.ShapeDtypeStruct((n_tokens, hidden), jnp.float32),
        scratch_types=[
            pltpu.VMEM((rpc, IDXW), jnp.int32),
            pltpu.VMEM((chunk_tokens, hidden), jnp.float32),
            pltpu.SemaphoreType.DMA,
        ],
    )
    def gather_kernel(table_hbm, idx_hbm, out_hbm, idx_v, rows_v, sem):
        wid = lax.axis_index("s") * nc + lax.axis_index("c")
        row0 = wid * rows_per_w

        def chunk(i, carry):
            r = row0 + i * rpc
            pltpu.sync_copy(idx_hbm.at[pl.ds(r, rpc)], idx_v)
            copies = []
            for j in range(rpc):
                copies.append(
                    pltpu.async_copy(
                        table_hbm.at[idx_v.at[j]],
                        rows_v.at[pl.ds(j * IDXW, IDXW)],
                        sem,
                    )
                )
            for c in copies:
                c.wait()
            pltpu.sync_copy(rows_v, out_hbm.at[pl.ds(r * IDXW, chunk_tokens)])
            return carry

        lax.fori_loop(0, n_chunks, chunk, 0)

    return gather_kernel


def kernel(input_ids, table):
    b, l = input_ids.shape
    vocab, hidden = table.shape
    n_tokens = b * l
    idx = input_ids.reshape(n_tokens // IDXW, IDXW).astype(jnp.int32)
    fn = _build(n_tokens, vocab, hidden)
    out = fn(table, idx)
    return out.reshape(b, l, hidden)


# speedup vs baseline: 1.0468x; 1.0468x over previous
import functools

import jax
import jax.numpy as jnp
from jax import lax
from jax.experimental import pallas as pl
from jax.experimental.pallas import tpu as pltpu
from jax.experimental.pallas import tpu_sc as plsc

IDXW = 128
RPC = 4


@functools.lru_cache(maxsize=None)
def _build(n_tokens: int, vocab: int, hidden: int):
    info = plsc.get_sparse_core_info()
    nc, ns = info.num_cores, info.num_subcores
    nw = nc * ns

    n_rows = n_tokens // IDXW
    rows_per_w = n_rows // nw
    n_chunks = rows_per_w // RPC
    chunk_tokens = RPC * IDXW

    assert n_rows % nw == 0 and rows_per_w % RPC == 0 and n_chunks % 2 == 0

    mesh = plsc.VectorSubcoreMesh(core_axis_name="c", subcore_axis_name="s")

    @functools.partial(
        pl.kernel,
        mesh=mesh,
        compiler_params=pltpu.CompilerParams(use_tc_tiling_on_sc=False),
        out_type=jax.ShapeDtypeStruct((n_tokens, hidden), jnp.float32),
        scratch_types=[
            pltpu.VMEM((rows_per_w, IDXW), jnp.int32),
            pltpu.VMEM((chunk_tokens, hidden), jnp.float32),
            pltpu.VMEM((chunk_tokens, hidden), jnp.float32),
            pltpu.SemaphoreType.DMA,
            pltpu.SemaphoreType.DMA,
            pltpu.SemaphoreType.DMA,
            pltpu.SemaphoreType.DMA,
        ],
    )
    def gather_kernel(table_hbm, idx_hbm, out_hbm, idx_all, rows0, rows1,
                      gsem0, gsem1, ssem0, ssem1):
        wid = lax.axis_index("s") * nc + lax.axis_index("c")
        row0 = wid * rows_per_w
        tok0 = row0 * IDXW

        rows = (rows0, rows1)
        gsem = (gsem0, gsem1)
        ssem = (ssem0, ssem1)

        pltpu.sync_copy(idx_hbm.at[pl.ds(row0, rows_per_w)], idx_all)

        def fire_gathers(i, b):
            for j in range(RPC):
                pltpu.async_copy(
                    table_hbm.at[idx_all.at[i * RPC + j]],
                    rows[b].at[pl.ds(j * IDXW, IDXW)],
                    gsem[b],
                )

        def wait_gathers(b):
            pltpu.make_async_copy(
                out_hbm.at[pl.ds(0, chunk_tokens)], rows[b], gsem[b]
            ).wait()

        def fire_store(i, b):
            pltpu.async_copy(
                rows[b],
                out_hbm.at[pl.ds(tok0 + i * chunk_tokens, chunk_tokens)],
                ssem[b],
            )

        def wait_store(b):
            pltpu.make_async_copy(
                rows[b], out_hbm.at[pl.ds(0, chunk_tokens)], ssem[b]
            ).wait()

        fire_gathers(0, 0)
        fire_gathers(1, 1)
        wait_gathers(0)
        fire_store(0, 0)

        def step(s, carry):
            i0 = s * 2
            wait_store(0)
            fire_gathers(i0, 0)
            wait_gathers(1)
            fire_store(i0 - 1, 1)
            wait_store(1)
            fire_gathers(i0 + 1, 1)
            wait_gathers(0)
            fire_store(i0, 0)
            return carry

        lax.fori_loop(1, n_chunks // 2, step, 0)

        wait_gathers(1)
        fire_store(n_chunks - 1, 1)
        wait_store(0)
        wait_store(1)

    return gather_kernel


def kernel(input_ids, table):
    b, l = input_ids.shape
    vocab, hidden = table.shape
    n_tokens = b * l
    idx = input_ids.reshape(n_tokens // IDXW, IDXW).astype(jnp.int32)
    fn = _build(n_tokens, vocab, hidden)
    out = fn(table, idx)
    return out.reshape(b, l, hidden)
